# baseline (speedup 1.0000x reference)
import functools

import jax
import jax.numpy as jnp
from jax import lax
from jax.experimental import pallas as pl
from jax.experimental.pallas import tpu as pltpu
from jax.experimental.pallas import tpu_sc as plsc

_R = 16384
_C = 4096
_NC = 2
_NS = 16
_NW = _NC * _NS
_CHUNK_ROWS = 4
_NBUF = 4


def _prune_chunk(buf):
    iota = lax.iota(jnp.int32, 16)
    base0 = iota * 4
    base1 = base0 + 1
    base2 = base0 + 2
    base3 = base0 + 3
    zero = jnp.zeros((16,), jnp.float32)

    for r in range(_CHUNK_ROWS):

        @plsc.parallel_loop(0, _C, step=64, unroll=4)
        def blk(o):
            b = buf.at[r, pl.ds(o, 64)]
            e0 = plsc.load_gather(b, [base0])
            e1 = plsc.load_gather(b, [base1])
            e2 = plsc.load_gather(b, [base2])
            e3 = plsc.load_gather(b, [base3])
            a0 = jnp.abs(e0)
            a1 = jnp.abs(e1)
            a2 = jnp.abs(e2)
            a3 = jnp.abs(e3)
            g10 = (a1 > a0).astype(jnp.int32)
            g20 = (a2 > a0).astype(jnp.int32)
            g30 = (a3 > a0).astype(jnp.int32)
            g21 = (a2 > a1).astype(jnp.int32)
            g31 = (a3 > a1).astype(jnp.int32)
            g32 = (a3 > a2).astype(jnp.int32)
            c0 = g10 + g20 + g30
            c1 = (1 - g10) + g21 + g31
            c2 = (2 - g20 - g21) + g32
            c3 = 3 - g30 - g31 - g32
            plsc.store_scatter(b, [base0], zero, mask=c0 > 1)
            plsc.store_scatter(b, [base1], zero, mask=c1 > 1)
            plsc.store_scatter(b, [base2], zero, mask=c2 > 1)
            plsc.store_scatter(b, [base3], zero, mask=c3 > 1)


@functools.lru_cache(maxsize=None)
def _make_sc_prune(nrows):
    rows_per_w = nrows // _NW
    nchunks = rows_per_w // _CHUNK_ROWS
    ng = nchunks // _NBUF

    @functools.partial(
        pl.kernel,
        mesh=plsc.VectorSubcoreMesh(core_axis_name="c", subcore_axis_name="s"),
        out_type=jax.ShapeDtypeStruct((nrows, _C), jnp.float32),
        scratch_types=[
            [pltpu.VMEM((_CHUNK_ROWS, _C), jnp.float32) for _ in range(_NBUF)],
            [pltpu.SemaphoreType.DMA for _ in range(_NBUF)],
            [pltpu.SemaphoreType.DMA for _ in range(_NBUF)],
        ],
        compiler_params=pltpu.CompilerParams(needs_layout_passes=False),
    )
    def _prune_sc(w_hbm, out_hbm, bufs, in_sems, out_sems):
        wid = lax.axis_index("s") * _NC + lax.axis_index("c")
        base = wid * rows_per_w

        for b in range(2):
            pltpu.async_copy(
                w_hbm.at[pl.ds(base + b * _CHUNK_ROWS, _CHUNK_ROWS)],
                bufs[b],
                in_sems[b],
            )

        def g_body(g, _):
            for b in range(_NBUF):
                ci = g * _NBUF + b
                row0 = base + ci * _CHUNK_ROWS
                b2 = (b + 2) % _NBUF
                pltpu.make_async_copy(
                    w_hbm.at[pl.ds(row0, _CHUNK_ROWS)], bufs[b], in_sems[b]
                ).wait()
                _prune_chunk(bufs[b])
                pltpu.async_copy(
                    bufs[b],
                    out_hbm.at[pl.ds(row0, _CHUNK_ROWS)],
                    out_sems[b],
                )

                def _recycle(prev_ci, next_ci):
                    pltpu.make_async_copy(
                        bufs[b2],
                        out_hbm.at[
                            pl.ds(base + prev_ci * _CHUNK_ROWS, _CHUNK_ROWS)
                        ],
                        out_sems[b2],
                    ).wait()
                    pltpu.async_copy(
                        w_hbm.at[
                            pl.ds(base + next_ci * _CHUNK_ROWS, _CHUNK_ROWS)
                        ],
                        bufs[b2],
                        in_sems[b2],
                    )

                if b < 2:
                    @pl.when(g > 0)
                    def _():
                        _recycle(ci - 2, ci + 2)

                    @pl.when(g == 0)
                    def _():
                        pltpu.async_copy(
                            w_hbm.at[
                                pl.ds(
                                    base + (ci + 2) * _CHUNK_ROWS, _CHUNK_ROWS
                                )
                            ],
                            bufs[b2],
                            in_sems[b2],
                        )
                else:
                    @pl.when(g < ng - 1)
                    def _():
                        _recycle(ci - 2, ci + 2)

            return 0

        lax.fori_loop(0, ng, g_body, 0)

        for ci in range(nchunks - _NBUF, nchunks):
            pltpu.make_async_copy(
                bufs[ci % _NBUF],
                out_hbm.at[pl.ds(base + ci * _CHUNK_ROWS, _CHUNK_ROWS)],
                out_sems[ci % _NBUF],
            ).wait()

    return _prune_sc


_SC_ROWS = 7168
_BR = 256
_BC = 128


def _tc_body(x_ref, o_ref):
    x = x_ref[...]
    a = jnp.abs(x)
    p = jax.lax.broadcasted_iota(jnp.int32, x.shape, 1) % 4
    cnt = jnp.zeros(x.shape, jnp.int32)
    for s in (1, 2, 3):
        fwd = pltpu.roll(a, x.shape[1] - s, 1)
        bwd = pltpu.roll(a, s, 1)
        vf = p + s <= 3
        vb = p - s >= 0
        cnt = cnt + jnp.where(vf & (fwd > a), 1, 0)
        cnt = cnt + jnp.where(vb & (bwd >= a), 1, 0)
    o_ref[...] = jnp.where(cnt < 2, x, 0.0)


def _tc_prune_tail(w):
    row0 = _SC_ROWS // _BR
    return pl.pallas_call(
        _tc_body,
        grid=((_R - _SC_ROWS) // _BR, _C // _BC),
        in_specs=[pl.BlockSpec((_BR, _BC), lambda i, j: (i + row0, j))],
        out_specs=pl.BlockSpec((_BR, _BC), lambda i, j: (i + row0, j)),
        out_shape=jax.ShapeDtypeStruct((_R, _C), jnp.float32),
    )(w)


@jax.jit
def kernel(weight):
    sc_part = _make_sc_prune(_SC_ROWS)(weight)
    tc_full = _tc_prune_tail(weight)
    return lax.dynamic_update_slice(tc_full, sc_part, (0, 0))


# speedup vs baseline: 3.1681x; 3.1681x over previous
import functools

import jax
import jax.numpy as jnp
from jax import lax
from jax.experimental import pallas as pl
from jax.experimental.pallas import tpu as pltpu
from jax.experimental.pallas import tpu_sc as plsc

_R = 16384
_C = 4096
_NC = 2
_NS = 16
_NW = _NC * _NS
_CHUNK_ROWS = 4
_NBUF = 4


def _prune_chunk(buf):
    iota = lax.iota(jnp.int32, 16)
    base0 = iota * 4
    base1 = base0 + 1
    base2 = base0 + 2
    base3 = base0 + 3
    zero = jnp.zeros((16,), jnp.float32)

    for r in range(_CHUNK_ROWS):

        @plsc.parallel_loop(0, _C, step=64, unroll=4)
        def blk(o):
            b = buf.at[r, pl.ds(o, 64)]
            e0 = plsc.load_gather(b, [base0])
            e1 = plsc.load_gather(b, [base1])
            e2 = plsc.load_gather(b, [base2])
            e3 = plsc.load_gather(b, [base3])
            a0 = jnp.abs(e0)
            a1 = jnp.abs(e1)
            a2 = jnp.abs(e2)
            a3 = jnp.abs(e3)
            g10 = (a1 > a0).astype(jnp.int32)
            g20 = (a2 > a0).astype(jnp.int32)
            g30 = (a3 > a0).astype(jnp.int32)
            g21 = (a2 > a1).astype(jnp.int32)
            g31 = (a3 > a1).astype(jnp.int32)
            g32 = (a3 > a2).astype(jnp.int32)
            c0 = g10 + g20 + g30
            c1 = (1 - g10) + g21 + g31
            c2 = (2 - g20 - g21) + g32
            c3 = 3 - g30 - g31 - g32
            plsc.store_scatter(b, [base0], zero, mask=c0 > 1)
            plsc.store_scatter(b, [base1], zero, mask=c1 > 1)
            plsc.store_scatter(b, [base2], zero, mask=c2 > 1)
            plsc.store_scatter(b, [base3], zero, mask=c3 > 1)


@functools.lru_cache(maxsize=None)
def _make_sc_prune(nrows):
    rows_per_w = nrows // _NW
    nchunks = rows_per_w // _CHUNK_ROWS
    ng = nchunks // _NBUF

    @functools.partial(
        pl.kernel,
        mesh=plsc.VectorSubcoreMesh(core_axis_name="c", subcore_axis_name="s"),
        out_type=jax.ShapeDtypeStruct((_R, _C), jnp.float32),
        scratch_types=[
            [pltpu.VMEM((_CHUNK_ROWS, _C), jnp.float32) for _ in range(_NBUF)],
            [pltpu.SemaphoreType.DMA for _ in range(_NBUF)],
            [pltpu.SemaphoreType.DMA for _ in range(_NBUF)],
        ],
        compiler_params=pltpu.CompilerParams(needs_layout_passes=False),
    )
    def _prune_sc(w_hbm, out_hbm, bufs, in_sems, out_sems):
        wid = lax.axis_index("s") * _NC + lax.axis_index("c")
        base = wid * rows_per_w

        for b in range(2):
            pltpu.async_copy(
                w_hbm.at[pl.ds(base + b * _CHUNK_ROWS, _CHUNK_ROWS)],
                bufs[b],
                in_sems[b],
            )

        def g_body(g, _):
            for b in range(_NBUF):
                ci = g * _NBUF + b
                row0 = base + ci * _CHUNK_ROWS
                b2 = (b + 2) % _NBUF
                pltpu.make_async_copy(
                    w_hbm.at[pl.ds(row0, _CHUNK_ROWS)], bufs[b], in_sems[b]
                ).wait()
                _prune_chunk(bufs[b])
                pltpu.async_copy(
                    bufs[b],
                    out_hbm.at[pl.ds(row0, _CHUNK_ROWS)],
                    out_sems[b],
                )

                def _recycle(prev_ci, next_ci):
                    pltpu.make_async_copy(
                        bufs[b2],
                        out_hbm.at[
                            pl.ds(base + prev_ci * _CHUNK_ROWS, _CHUNK_ROWS)
                        ],
                        out_sems[b2],
                    ).wait()
                    pltpu.async_copy(
                        w_hbm.at[
                            pl.ds(base + next_ci * _CHUNK_ROWS, _CHUNK_ROWS)
                        ],
                        bufs[b2],
                        in_sems[b2],
                    )

                if b < 2:
                    @pl.when(g > 0)
                    def _():
                        _recycle(ci - 2, ci + 2)

                    @pl.when(g == 0)
                    def _():
                        pltpu.async_copy(
                            w_hbm.at[
                                pl.ds(
                                    base + (ci + 2) * _CHUNK_ROWS, _CHUNK_ROWS
                                )
                            ],
                            bufs[b2],
                            in_sems[b2],
                        )
                else:
                    @pl.when(g < ng - 1)
                    def _():
                        _recycle(ci - 2, ci + 2)

            return 0

        lax.fori_loop(0, ng, g_body, 0)

        for ci in range(nchunks - _NBUF, nchunks):
            pltpu.make_async_copy(
                bufs[ci % _NBUF],
                out_hbm.at[pl.ds(base + ci * _CHUNK_ROWS, _CHUNK_ROWS)],
                out_sems[ci % _NBUF],
            ).wait()

    return _prune_sc


_SC_ROWS = 11264
_BR = 256


def _tc_body(x_ref, o_ref):
    x = x_ref[...]
    a = jnp.abs(x)
    p = jax.lax.broadcasted_iota(jnp.int32, x.shape, 1) % 4
    cnt = jnp.zeros(x.shape, jnp.int32)
    for s in (1, 2, 3):
        fwd = pltpu.roll(a, x.shape[1] - s, 1)
        bwd = pltpu.roll(a, s, 1)
        vf = p + s <= 3
        vb = p - s >= 0
        cnt = cnt + jnp.where(vf & (fwd > a), 1, 0)
        cnt = cnt + jnp.where(vb & (bwd >= a), 1, 0)
    o_ref[...] = jnp.where(cnt < 2, x, 0.0)


def _tc_prune_tail(w):
    row0 = _SC_ROWS // _BR
    return pl.pallas_call(
        _tc_body,
        grid=((_R - _SC_ROWS) // _BR,),
        in_specs=[pl.BlockSpec((_BR, _C), lambda i: (i + row0, 0))],
        out_specs=pl.BlockSpec((_BR, _C), lambda i: (i, 0)),
        out_shape=jax.ShapeDtypeStruct((_R - _SC_ROWS, _C), jnp.float32),
    )(w)


@jax.jit
def kernel(weight):
    sc_full = _make_sc_prune(_SC_ROWS)(weight)
    tc_part = _tc_prune_tail(weight)
    return lax.dynamic_update_slice(sc_full, tc_part, (_SC_ROWS, 0))


# speedup vs baseline: 3.1832x; 1.0048x over previous
import functools

import jax
import jax.numpy as jnp
from jax import lax
from jax.experimental import pallas as pl
from jax.experimental.pallas import tpu as pltpu
from jax.experimental.pallas import tpu_sc as plsc

_R = 16384
_C = 4096
_NC = 2
_NS = 16
_NW = _NC * _NS
_CHUNK_ROWS = 4
_NBUF = 4


def _prune_chunk(buf):
    iota = lax.iota(jnp.int32, 16)
    base0 = iota * 4
    base1 = base0 + 1
    base2 = base0 + 2
    base3 = base0 + 3
    zero = jnp.zeros((16,), jnp.float32)

    for r in range(_CHUNK_ROWS):

        @plsc.parallel_loop(0, _C, step=64, unroll=4)
        def blk(o):
            b = buf.at[r, pl.ds(o, 64)]
            e0 = plsc.load_gather(b, [base0])
            e1 = plsc.load_gather(b, [base1])
            e2 = plsc.load_gather(b, [base2])
            e3 = plsc.load_gather(b, [base3])
            a0 = jnp.abs(e0)
            a1 = jnp.abs(e1)
            a2 = jnp.abs(e2)
            a3 = jnp.abs(e3)
            g10 = (a1 > a0).astype(jnp.int32)
            g20 = (a2 > a0).astype(jnp.int32)
            g30 = (a3 > a0).astype(jnp.int32)
            g21 = (a2 > a1).astype(jnp.int32)
            g31 = (a3 > a1).astype(jnp.int32)
            g32 = (a3 > a2).astype(jnp.int32)
            c0 = g10 + g20 + g30
            c1 = (1 - g10) + g21 + g31
            c2 = (2 - g20 - g21) + g32
            c3 = 3 - g30 - g31 - g32
            plsc.store_scatter(b, [base0], zero, mask=c0 > 1)
            plsc.store_scatter(b, [base1], zero, mask=c1 > 1)
            plsc.store_scatter(b, [base2], zero, mask=c2 > 1)
            plsc.store_scatter(b, [base3], zero, mask=c3 > 1)


@functools.lru_cache(maxsize=None)
def _make_sc_prune(nrows):
    rows_per_w = nrows // _NW
    nchunks = rows_per_w // _CHUNK_ROWS
    ng = nchunks // _NBUF

    @functools.partial(
        pl.kernel,
        mesh=plsc.VectorSubcoreMesh(core_axis_name="c", subcore_axis_name="s"),
        out_type=jax.ShapeDtypeStruct((_R, _C), jnp.float32),
        scratch_types=[
            [pltpu.VMEM((_CHUNK_ROWS, _C), jnp.float32) for _ in range(_NBUF)],
            [pltpu.SemaphoreType.DMA for _ in range(_NBUF)],
            [pltpu.SemaphoreType.DMA for _ in range(_NBUF)],
        ],
        compiler_params=pltpu.CompilerParams(needs_layout_passes=False),
    )
    def _prune_sc(w_hbm, out_hbm, bufs, in_sems, out_sems):
        wid = lax.axis_index("s") * _NC + lax.axis_index("c")
        base = wid * rows_per_w

        for b in range(2):
            pltpu.async_copy(
                w_hbm.at[pl.ds(base + b * _CHUNK_ROWS, _CHUNK_ROWS)],
                bufs[b],
                in_sems[b],
            )

        def g_body(g, _):
            for b in range(_NBUF):
                ci = g * _NBUF + b
                row0 = base + ci * _CHUNK_ROWS
                b2 = (b + 2) % _NBUF
                pltpu.make_async_copy(
                    w_hbm.at[pl.ds(row0, _CHUNK_ROWS)], bufs[b], in_sems[b]
                ).wait()
                _prune_chunk(bufs[b])
                pltpu.async_copy(
                    bufs[b],
                    out_hbm.at[pl.ds(row0, _CHUNK_ROWS)],
                    out_sems[b],
                )

                def _recycle(prev_ci, next_ci):
                    pltpu.make_async_copy(
                        bufs[b2],
                        out_hbm.at[
                            pl.ds(base + prev_ci * _CHUNK_ROWS, _CHUNK_ROWS)
                        ],
                        out_sems[b2],
                    ).wait()
                    pltpu.async_copy(
                        w_hbm.at[
                            pl.ds(base + next_ci * _CHUNK_ROWS, _CHUNK_ROWS)
                        ],
                        bufs[b2],
                        in_sems[b2],
                    )

                if b < 2:
                    @pl.when(g > 0)
                    def _():
                        _recycle(ci - 2, ci + 2)

                    @pl.when(g == 0)
                    def _():
                        pltpu.async_copy(
                            w_hbm.at[
                                pl.ds(
                                    base + (ci + 2) * _CHUNK_ROWS, _CHUNK_ROWS
                                )
                            ],
                            bufs[b2],
                            in_sems[b2],
                        )
                else:
                    @pl.when(g < ng - 1)
                    def _():
                        _recycle(ci - 2, ci + 2)

            return 0

        lax.fori_loop(0, ng, g_body, 0)

        for ci in range(nchunks - _NBUF, nchunks):
            pltpu.make_async_copy(
                bufs[ci % _NBUF],
                out_hbm.at[pl.ds(base + ci * _CHUNK_ROWS, _CHUNK_ROWS)],
                out_sems[ci % _NBUF],
            ).wait()

    return _prune_sc


_SC_ROWS = 11264
_BR = 256


def _tc_body(x_ref, o_ref):
    x = x_ref[...]
    a = jnp.abs(x)
    p = jax.lax.broadcasted_iota(jnp.int32, x.shape, 1) % 4
    neg = jnp.full(x.shape, -1.0, jnp.float32)
    cnt = jnp.zeros(x.shape, jnp.int32)
    for s in (1, 2, 3):
        fwd = pltpu.roll(a, x.shape[1] - s, 1)
        bwd = pltpu.roll(a, s, 1)
        fwdm = jnp.where(p + s <= 3, fwd, neg)
        bwdm = jnp.where(p - s >= 0, bwd, neg)
        cnt = cnt + (fwdm > a) + (bwdm >= a)
    o_ref[...] = jnp.where(cnt < 2, x, 0.0)


def _tc_prune_tail(w):
    row0 = _SC_ROWS // _BR
    return pl.pallas_call(
        _tc_body,
        grid=((_R - _SC_ROWS) // _BR,),
        in_specs=[pl.BlockSpec((_BR, _C), lambda i: (i + row0, 0))],
        out_specs=pl.BlockSpec((_BR, _C), lambda i: (i, 0)),
        out_shape=jax.ShapeDtypeStruct((_R - _SC_ROWS, _C), jnp.float32),
    )(w)


@jax.jit
def kernel(weight):
    sc_full = _make_sc_prune(_SC_ROWS)(weight)
    tc_part = _tc_prune_tail(weight)
    return lax.dynamic_update_slice(sc_full, tc_part, (_SC_ROWS, 0))
